# baseline (speedup 1.0000x reference)
import numpy as np
import jax
import jax.numpy as jnp
from jax import lax
from jax.experimental import pallas as pl
from jax.experimental.pallas import tpu as pltpu

_WS = 2048
_HEADS = 4
_DH = 7
_SCALE = 28.0 ** -0.5
_QCHUNK = 512
_CONN_NORM = np.float32(1.0) / np.float32(np.sqrt(np.float32(2.0))) * np.float32(0.5)

_EDGE_NAMES = ["front", "rear", "right", "left"]
_EDGE_LIST = [([2, 3], [0, 1]), ([0, 1], [2, 3]), ([0, 2], [1, 3]), ([1, 3], [0, 2])]


def _adj(src, dst):
    a = np.zeros((4, 4), np.float32)
    for s, d in zip(src, dst):
        a[d, s] = 1.0
    return a


_A_W = np.stack([_adj(s, d) for s, d in _EDGE_LIST])

_VEC_LENS = [("few_b1", 20), ("few_b2", 15), ("c1bsum", 8), ("c2bsum", 4),
             ("c2cb", 4), ("ntb1", 10), ("ntb2", 5), ("rseb", 28), ("dmp", 7)]
_LAYER_VEC_LENS = [("ln1g", 28), ("ln1b", 28), ("ob", 28), ("ln2g", 28),
                   ("ln2b", 28), ("fb1", 14), ("fb2", 28)]
_VEC_OFF = {}
_off = 0
for _n, _l in _VEC_LENS:
    _VEC_OFF[_n] = (_off, _l)
    _off += _l
for _L in range(3):
    for _n, _l in _LAYER_VEC_LENS:
        _VEC_OFF[f"{_n}{_L}"] = (_off, _l)
        _off += _l
_VEC_TOTAL = _off


def _bf(t):
    return t.astype(jnp.bfloat16)


def _dot(a, b):
    return jnp.dot(_bf(a), _bf(b), preferred_element_type=jnp.float32)


def _dotb(a, b):
    return jnp.dot(_bf(a), b, preferred_element_type=jnp.float32)


def _layer_norm(x, g, b):
    mu = jnp.mean(x, axis=-1, keepdims=True)
    var = jnp.mean((x - mu) ** 2, axis=-1, keepdims=True)
    return (x - mu) / jnp.sqrt(var + 1e-5) * g + b


def _tile4(b):
    return jnp.concatenate([b, b, b, b], axis=1)


def _main_body(dist, wf, nt, bd1, bd2, m1, m2, c2c, ntw1, ntw2, rse,
               qkvw, ow, f1, f2, vecs, x_out):
    f32 = jnp.float32
    bf16 = jnp.bfloat16
    lr = lambda t: jnp.where(t >= 0, t, 0.01 * t)
    vp = vecs[...]

    def vec(name):
        off, ln = _VEC_OFF[name]
        return vp[:, off:off + ln]

    u = lr(_dotb(wf[...], bd1[...]) + _tile4(vec("few_b1")))
    w = _dotb(u, bd2[...]) + _tile4(vec("few_b2"))
    h1 = lr(_dotb(w, m1[...]) + _tile4(vec("c1bsum")))
    h2w = _dotb(h1, m2[...]) + _tile4(vec("c2bsum"))
    aggc = (h1[:, 0:8] + h1[:, 8:16] + h1[:, 16:24] + h1[:, 24:32]) * _CONN_NORM
    h2s = _dotb(aggc, c2c[...]) + vec("c2cb")
    ntv = jnp.maximum(_dotb(nt[...], ntw1[...]) + vec("ntb1"), 0.0)
    nt5 = _dotb(ntv, ntw2[...]) + vec("ntb2")
    x37 = jnp.concatenate(
        [dist[...], h2w, h2s, h2s, nt5,
         jnp.broadcast_to(vec("dmp"), (_WS, 7))], axis=1)
    x = _dotb(x37, rse[...]) + vec("rseb")

    for L in range(3):
        y = _layer_norm(x, vec(f"ln1g{L}"), vec(f"ln1b{L}"))
        qkv = _dotb(y, qkvw[L])
        outs = []
        for h in range(_HEADS):
            q = _bf(qkv[:, h * _DH:(h + 1) * _DH])
            k = _bf(qkv[:, 28 + h * _DH:28 + (h + 1) * _DH])
            v = _bf(qkv[:, 56 + h * _DH:56 + (h + 1) * _DH])
            och = []
            for t0 in range(0, _WS, _QCHUNK):
                dots = lax.dot_general(q[t0:t0 + _QCHUNK], k,
                                       (((1,), (1,)), ((), ())),
                                       preferred_element_type=f32)
                e = jnp.exp(dots)
                s = jnp.sum(e, axis=-1, keepdims=True)
                a = (e * (1.0 / s)).astype(bf16)
                och.append(jnp.dot(a, v, preferred_element_type=f32))
            outs.append(jnp.concatenate(och, axis=0))
        o = jnp.concatenate(outs, axis=1)
        x = x + _dotb(o, ow[L]) + vec(f"ob{L}")
        y2 = _layer_norm(x, vec(f"ln2g{L}"), vec(f"ln2b{L}"))
        g = _dotb(y2, f1[L]) + vec(f"fb1{L}")
        g = 0.5 * g * (1.0 + lax.erf(g / np.float32(np.sqrt(2.0))))
        x = x + _dotb(g, f2[L]) + vec(f"fb2{L}")

    x_out[...] = x


_HEAD_CHUNK = 256


def _head_body(x_ref, w1_ref, hb1, hw2, hb2, hw3, hb3, out_ref, g_ref):
    i = pl.program_id(0)
    f32 = jnp.float32

    @pl.when(i == 0)
    def _():
        g_ref[...] = jnp.zeros_like(g_ref)

    xb = _bf(x_ref[...]).astype(f32)
    xrep = jnp.repeat(xb, 128, axis=1)
    g_ref[...] += jnp.sum(xrep * w1_ref[...].astype(f32), axis=0,
                          keepdims=True)

    @pl.when(i == pl.num_programs(0) - 1)
    def _():
        g = g_ref[...]
        acc = jnp.zeros((1, 128), f32)
        for c in range(28):
            acc = acc + g[:, c * 128:(c + 1) * 128]
        r = jnp.maximum(acc + hb1[...].reshape(1, -1), 0.0)
        r = jnp.maximum(_dot(r, hw2[...]) + hb2[...].reshape(1, -1), 0.0)
        out_ref[...] = _dot(r, hw3[...]) + hb3[...].reshape(1, -1)


def kernel(distance, lane, wheel_feat, sensor_feat, norm_target, damper_idx,
           params):
    del lane, sensor_feat
    p = params
    f32 = jnp.float32
    bf16 = jnp.bfloat16
    eye4 = jnp.eye(4, dtype=f32)

    bd1 = jnp.kron(eye4, p["few_W1"]).astype(bf16)
    bd2 = jnp.kron(eye4, p["few_W2"]).astype(bf16)
    c1 = jnp.stack([p[f"c1_{e}_W"] for e in _EDGE_NAMES])
    m1 = jnp.einsum("eij,ekl->jkil", jnp.asarray(_A_W),
                    c1).reshape(60, 32).astype(bf16)
    c2 = jnp.stack([p[f"c2_{e}_W"] for e in _EDGE_NAMES])
    m2 = jnp.einsum("eij,ekl->jkil", jnp.asarray(_A_W),
                    c2).reshape(32, 16).astype(bf16)
    w1r = p["head_W1"].reshape(_WS, 28 * 128).astype(bf16)

    vec_parts = [p["few_b1"], p["few_b2"],
                 sum(p[f"c1_{e}_b"] for e in _EDGE_NAMES),
                 sum(p[f"c2_{e}_b"] for e in _EDGE_NAMES),
                 p["c2_connect_b"], p["nt_b1"], p["nt_b2"], p["rse_b"],
                 p["damper_E"][damper_idx]]
    for L in p["layers"]:
        vec_parts += [L["ln1_g"], L["ln1_b"], L["out_b"], L["ln2_g"],
                      L["ln2_b"], L["ff_b1"], L["ff_b2"]]
    vecs = jnp.concatenate(vec_parts)[None, :]

    qscale = jnp.concatenate([jnp.full((28,), _SCALE, f32),
                              jnp.ones((56,), f32)])
    qkvw = (jnp.stack([L["qkv_W"] for L in p["layers"]])
            * qscale).astype(bf16)
    ow = jnp.stack([L["out_W"] for L in p["layers"]]).astype(bf16)
    f1 = jnp.stack([L["ff_W1"] for L in p["layers"]]).astype(bf16)
    f2 = jnp.stack([L["ff_W2"] for L in p["layers"]]).astype(bf16)

    wf160 = wheel_feat.reshape(_WS, 160)
    nt20 = norm_target.reshape(_WS, 20)

    x = pl.pallas_call(
        _main_body,
        out_shape=jax.ShapeDtypeStruct((_WS, 28), f32),
    )(distance, wf160, nt20, bd1, bd2, m1, m2,
      p["c2_connect_W"].astype(bf16), p["nt_W1"].astype(bf16),
      p["nt_W2"].astype(bf16), p["rse_W"].astype(bf16),
      qkvw, ow, f1, f2, vecs)

    n_chunks = _WS // _HEAD_CHUNK
    out = pl.pallas_call(
        _head_body,
        grid=(n_chunks,),
        in_specs=[
            pl.BlockSpec((_HEAD_CHUNK, 28), lambda i: (i, 0)),
            pl.BlockSpec((_HEAD_CHUNK, 28 * 128), lambda i: (i, 0)),
            pl.BlockSpec((128,), lambda i: (0,)),
            pl.BlockSpec((128, 32), lambda i: (0, 0)),
            pl.BlockSpec((32,), lambda i: (0,)),
            pl.BlockSpec((32, 4), lambda i: (0, 0)),
            pl.BlockSpec((4,), lambda i: (0,)),
        ],
        out_specs=pl.BlockSpec((1, 4), lambda i: (0, 0)),
        out_shape=jax.ShapeDtypeStruct((1, 4), f32),
        scratch_shapes=[pltpu.VMEM((1, 28 * 128), f32)],
    )(x, w1r, p["head_b1"], p["head_W2"], p["head_b2"], p["head_W3"],
      p["head_b3"])
    return out


# speedup vs baseline: 1.0277x; 1.0277x over previous
import numpy as np
import jax
import jax.numpy as jnp
from jax import lax
from jax.experimental import pallas as pl
from jax.experimental.pallas import tpu as pltpu

_WS = 2048
_HEADS = 4
_DH = 7
_SCALE = 28.0 ** -0.5
_QCHUNK = 512
_CONN_NORM = np.float32(1.0) / np.float32(np.sqrt(np.float32(2.0))) * np.float32(0.5)

_EDGE_NAMES = ["front", "rear", "right", "left"]
_EDGE_LIST = [([2, 3], [0, 1]), ([0, 1], [2, 3]), ([0, 2], [1, 3]), ([1, 3], [0, 2])]


def _adj(src, dst):
    a = np.zeros((4, 4), np.float32)
    for s, d in zip(src, dst):
        a[d, s] = 1.0
    return a


_A_W = np.stack([_adj(s, d) for s, d in _EDGE_LIST])

_VEC_LENS = [("few_b1", 20), ("few_b2", 15), ("c1bsum", 8), ("c2bsum", 4),
             ("c2cb", 4), ("ntb1", 10), ("ntb2", 5), ("rseb", 28), ("dmp", 7)]
_LAYER_VEC_LENS = [("ln1g", 28), ("ln1b", 28), ("ob", 28), ("ln2g", 28),
                   ("ln2b", 28), ("fb1", 14), ("fb2", 28)]
_VEC_OFF = {}
_off = 0
for _n, _l in _VEC_LENS:
    _VEC_OFF[_n] = (_off, _l)
    _off += _l
for _L in range(3):
    for _n, _l in _LAYER_VEC_LENS:
        _VEC_OFF[f"{_n}{_L}"] = (_off, _l)
        _off += _l
_VEC_TOTAL = _off


def _bf(t):
    return t.astype(jnp.bfloat16)


def _dot(a, b):
    return jnp.dot(_bf(a), _bf(b), preferred_element_type=jnp.float32)


def _dotb(a, b):
    return jnp.dot(_bf(a), b, preferred_element_type=jnp.float32)


def _layer_norm(x, g, b):
    mu = jnp.mean(x, axis=-1, keepdims=True)
    var = jnp.mean((x - mu) ** 2, axis=-1, keepdims=True)
    return (x - mu) / jnp.sqrt(var + 1e-5) * g + b


def _tile4(b):
    return jnp.concatenate([b, b, b, b], axis=1)


def _main_body(dist, wf, nt, bd1, bd2, m1, m2, c2c, ntw1, ntw2, rse,
               qkvw, ow, f1, f2, vecs, x_out):
    f32 = jnp.float32
    bf16 = jnp.bfloat16
    lr = lambda t: jnp.where(t >= 0, t, 0.01 * t)
    vp = vecs[...]

    def vec(name):
        off, ln = _VEC_OFF[name]
        return vp[:, off:off + ln]

    u = lr(_dotb(wf[...], bd1[...]) + _tile4(vec("few_b1")))
    w = _dotb(u, bd2[...]) + _tile4(vec("few_b2"))
    h1 = lr(_dotb(w, m1[...]) + _tile4(vec("c1bsum")))
    h2w = _dotb(h1, m2[...]) + _tile4(vec("c2bsum"))
    aggc = (h1[:, 0:8] + h1[:, 8:16] + h1[:, 16:24] + h1[:, 24:32]) * _CONN_NORM
    h2s = _dotb(aggc, c2c[...]) + vec("c2cb")
    ntv = jnp.maximum(_dotb(nt[...], ntw1[...]) + vec("ntb1"), 0.0)
    nt5 = _dotb(ntv, ntw2[...]) + vec("ntb2")
    x37 = jnp.concatenate(
        [dist[...], h2w, h2s, h2s, nt5,
         jnp.broadcast_to(vec("dmp"), (_WS, 7))], axis=1)
    x = _dotb(x37, rse[...]) + vec("rseb")

    for L in range(3):
        y = _layer_norm(x, vec(f"ln1g{L}"), vec(f"ln1b{L}"))
        qkv = _dotb(y, qkvw[L])
        outs = []
        for h in range(_HEADS):
            q = _bf(qkv[:, h * _DH:(h + 1) * _DH])
            k = _bf(qkv[:, 28 + h * _DH:28 + (h + 1) * _DH])
            v = _bf(qkv[:, 56 + h * _DH:56 + (h + 1) * _DH])
            och = []
            for t0 in range(0, _WS, _QCHUNK):
                dots = lax.dot_general(q[t0:t0 + _QCHUNK], k,
                                       (((1,), (1,)), ((), ())),
                                       preferred_element_type=f32)
                e = jnp.exp(dots)
                s = jnp.sum(e, axis=-1, keepdims=True)
                a = (e * (1.0 / s)).astype(bf16)
                och.append(jnp.dot(a, v, preferred_element_type=f32))
            outs.append(jnp.concatenate(och, axis=0))
        o = jnp.concatenate(outs, axis=1)
        x = x + _dotb(o, ow[L]) + vec(f"ob{L}")
        y2 = _layer_norm(x, vec(f"ln2g{L}"), vec(f"ln2b{L}"))
        g = _dotb(y2, f1[L]) + vec(f"fb1{L}")
        g = 0.5 * g * (1.0 + lax.erf(g / np.float32(np.sqrt(2.0))))
        x = x + _dotb(g, f2[L]) + vec(f"fb2{L}")

    x_out[...] = x


_HEAD_CHUNK = 256


def _head_body(x_ref, w1_ref, hb1, hw2, hb2, hw3, hb3, out_ref, g_ref):
    i = pl.program_id(0)
    f32 = jnp.float32

    @pl.when(i == 0)
    def _():
        g_ref[...] = jnp.zeros_like(g_ref)

    xt = _bf(x_ref[...]).T
    g_ref[...] += jnp.dot(xt, w1_ref[...], preferred_element_type=f32)

    @pl.when(i == pl.num_programs(0) - 1)
    def _():
        g = g_ref[...]
        acc = jnp.zeros((1, 128), f32)
        for c in range(28):
            acc = acc + g[c:c + 1, c * 128:(c + 1) * 128]
        r = jnp.maximum(acc + hb1[...].reshape(1, -1), 0.0)
        r = jnp.maximum(_dot(r, hw2[...]) + hb2[...].reshape(1, -1), 0.0)
        out_ref[...] = _dot(r, hw3[...]) + hb3[...].reshape(1, -1)


def kernel(distance, lane, wheel_feat, sensor_feat, norm_target, damper_idx,
           params):
    del lane, sensor_feat
    p = params
    f32 = jnp.float32
    bf16 = jnp.bfloat16
    eye4 = jnp.eye(4, dtype=f32)

    bd1 = jnp.kron(eye4, p["few_W1"]).astype(bf16)
    bd2 = jnp.kron(eye4, p["few_W2"]).astype(bf16)
    c1 = jnp.stack([p[f"c1_{e}_W"] for e in _EDGE_NAMES])
    m1 = jnp.einsum("eij,ekl->jkil", jnp.asarray(_A_W),
                    c1).reshape(60, 32).astype(bf16)
    c2 = jnp.stack([p[f"c2_{e}_W"] for e in _EDGE_NAMES])
    m2 = jnp.einsum("eij,ekl->jkil", jnp.asarray(_A_W),
                    c2).reshape(32, 16).astype(bf16)
    w1r = p["head_W1"].astype(bf16).reshape(_WS, 28 * 128)

    vec_parts = [p["few_b1"], p["few_b2"],
                 sum(p[f"c1_{e}_b"] for e in _EDGE_NAMES),
                 sum(p[f"c2_{e}_b"] for e in _EDGE_NAMES),
                 p["c2_connect_b"], p["nt_b1"], p["nt_b2"], p["rse_b"],
                 p["damper_E"][damper_idx]]
    for L in p["layers"]:
        vec_parts += [L["ln1_g"], L["ln1_b"], L["out_b"], L["ln2_g"],
                      L["ln2_b"], L["ff_b1"], L["ff_b2"]]
    vecs = jnp.concatenate(vec_parts)[None, :]

    qscale = jnp.concatenate([jnp.full((28,), _SCALE, f32),
                              jnp.ones((56,), f32)])
    qkvw = (jnp.stack([L["qkv_W"] for L in p["layers"]])
            * qscale).astype(bf16)
    ow = jnp.stack([L["out_W"] for L in p["layers"]]).astype(bf16)
    f1 = jnp.stack([L["ff_W1"] for L in p["layers"]]).astype(bf16)
    f2 = jnp.stack([L["ff_W2"] for L in p["layers"]]).astype(bf16)

    wf160 = wheel_feat.reshape(_WS, 160)
    nt20 = norm_target.reshape(_WS, 20)

    x = pl.pallas_call(
        _main_body,
        out_shape=jax.ShapeDtypeStruct((_WS, 28), f32),
    )(distance, wf160, nt20, bd1, bd2, m1, m2,
      p["c2_connect_W"].astype(bf16), p["nt_W1"].astype(bf16),
      p["nt_W2"].astype(bf16), p["rse_W"].astype(bf16),
      qkvw, ow, f1, f2, vecs)

    n_chunks = _WS // _HEAD_CHUNK
    out = pl.pallas_call(
        _head_body,
        grid=(n_chunks,),
        in_specs=[
            pl.BlockSpec((_HEAD_CHUNK, 28), lambda i: (i, 0)),
            pl.BlockSpec((_HEAD_CHUNK, 28 * 128), lambda i: (i, 0)),
            pl.BlockSpec((128,), lambda i: (0,)),
            pl.BlockSpec((128, 32), lambda i: (0, 0)),
            pl.BlockSpec((32,), lambda i: (0,)),
            pl.BlockSpec((32, 4), lambda i: (0, 0)),
            pl.BlockSpec((4,), lambda i: (0,)),
        ],
        out_specs=pl.BlockSpec((1, 4), lambda i: (0, 0)),
        out_shape=jax.ShapeDtypeStruct((1, 4), f32),
        scratch_shapes=[pltpu.VMEM((28, 28 * 128), f32)],
    )(x, w1r, p["head_b1"], p["head_W2"], p["head_b2"], p["head_W3"],
      p["head_b3"])
    return out
